# Initial kernel scaffold; baseline (speedup 1.0000x reference)
import jax
import jax.numpy as jnp
from jax.experimental import pallas as pl


def kernel(x, y):
    raise NotImplementedError("write your pallas kernel here")


# speedup vs baseline: 25.3026x; 25.3026x over previous
import jax
import jax.numpy as jnp
from jax.experimental import pallas as pl
from jax.experimental.pallas import tpu as pltpu

_B = 16384
_C = 1000
_BR = 512
_NB = _B // _BR

_NEG = float(-jnp.inf)


def _dlr_body(x_ref, y_ref, out_ref):
    xb = x_ref[...]
    yv = y_ref[0]

    m1 = jnp.max(xb, axis=1, keepdims=True)
    c1 = xb == m1
    k1 = jnp.sum(c1.astype(jnp.float32), axis=1, keepdims=True)
    u = jnp.max(jnp.where(c1, _NEG, xb), axis=1, keepdims=True)
    c2 = xb == u
    k2 = jnp.sum(c2.astype(jnp.float32), axis=1, keepdims=True)
    v = jnp.max(jnp.where(c1 | c2, _NEG, xb), axis=1, keepdims=True)

    m2 = jnp.where(k1 > 1.0, m1, u)
    m3 = jnp.where(k1 > 2.0, m1, jnp.where(k1 + k2 > 2.0, u, v))

    col = jax.lax.broadcasted_iota(jnp.int32, (_BR, _C), 1)
    true_logit = jnp.max(jnp.where(col == yv, xb, _NEG), axis=1, keepdims=True)

    ind = true_logit == m1
    dividend = true_logit - jnp.where(ind, m2, m1)
    divisor = m1 - m3 + 1e-12
    out_ref[0] = jnp.sum(dividend / divisor, axis=0, keepdims=True)


def kernel(x, y):
    y3 = y.astype(jnp.int32).reshape(_NB, _BR, 1)
    partial = pl.pallas_call(
        _dlr_body,
        grid=(_NB,),
        in_specs=[
            pl.BlockSpec((_BR, _C), lambda i: (i, 0)),
            pl.BlockSpec((1, _BR, 1), lambda i: (i, 0, 0)),
        ],
        out_specs=pl.BlockSpec((1, 1, 1), lambda i: (i, 0, 0)),
        out_shape=jax.ShapeDtypeStruct((_NB, 1, 1), jnp.float32),
        compiler_params=pltpu.CompilerParams(
            dimension_semantics=("parallel",),
        ),
    )(x, y3)
    return -(jnp.sum(partial) / _B)


# speedup vs baseline: 26.1464x; 1.0333x over previous
import jax
import jax.numpy as jnp
from jax.experimental import pallas as pl
from jax.experimental.pallas import tpu as pltpu

_B = 16384
_C = 1000
_BR = 512
_NB = _B // _BR

_NEG = float(-jnp.inf)


def _dlr_body(x_ref, y_ref, out_ref):
    xb = x_ref[...]
    yv = y_ref[0]

    gs = [xb[:, k * 128:(k + 1) * 128] for k in range(7)]
    lane = jax.lax.broadcasted_iota(jnp.int32, (_BR, 128), 1)
    gs.append(jnp.where(lane >= 24, xb[:, 872:1000], _NEG))

    hi = [jnp.maximum(gs[2 * i], gs[2 * i + 1]) for i in range(4)]
    lo = [jnp.minimum(gs[2 * i], gs[2 * i + 1]) for i in range(4)]

    def top3of4(a, b, c, d):
        p1 = jnp.maximum(a, c)
        p2 = jnp.minimum(a, c)
        q1 = jnp.maximum(b, d)
        return p1, jnp.maximum(p2, q1), jnp.minimum(p2, q1)

    x1, x2, x3 = top3of4(hi[0], lo[0], hi[1], lo[1])
    y1, y2, y3 = top3of4(hi[2], lo[2], hi[3], lo[3])
    z1 = jnp.maximum(x1, y1)
    m11 = jnp.minimum(x1, y1)
    m22 = jnp.maximum(x2, y2)
    z2 = jnp.maximum(m11, m22)
    z3 = jnp.maximum(jnp.maximum(jnp.minimum(m22, m11), jnp.minimum(x2, y2)),
                     jnp.maximum(x3, y3))

    cand = jnp.concatenate([z2, z3], axis=1)
    m1 = jnp.max(z1, axis=1, keepdims=True)
    c1z = z1 == m1
    c1c = cand == m1
    k1 = (jnp.sum(jnp.where(c1z, 1.0, 0.0), axis=1, keepdims=True)
          + jnp.sum(jnp.where(c1c, 1.0, 0.0), axis=1, keepdims=True))
    uz = jnp.where(c1z, _NEG, z1)
    uc = jnp.where(c1c, _NEG, cand)
    u = jnp.maximum(jnp.max(uz, axis=1, keepdims=True),
                    jnp.max(uc, axis=1, keepdims=True))
    c2z = uz == u
    c2c = uc == u
    k2 = (jnp.sum(jnp.where(c2z, 1.0, 0.0), axis=1, keepdims=True)
          + jnp.sum(jnp.where(c2c, 1.0, 0.0), axis=1, keepdims=True))
    v = jnp.maximum(
        jnp.max(jnp.where(c2z, _NEG, uz), axis=1, keepdims=True),
        jnp.max(jnp.where(c2c, _NEG, uc), axis=1, keepdims=True))
    m2 = jnp.where(k1 > 1.0, m1, u)
    m3 = jnp.where(k1 > 2.0, m1, jnp.where(k1 + k2 > 2.0, u, v))

    col = jax.lax.broadcasted_iota(jnp.int32, (_BR, _C), 1)
    tl = jnp.max(jnp.where(col == yv, xb, _NEG), axis=1, keepdims=True)

    ind = tl == m1
    dividend = tl - jnp.where(ind, m2, m1)
    divisor = m1 - m3 + 1e-12
    out_ref[0] = jnp.sum(dividend / divisor, axis=0, keepdims=True)


def kernel(x, y):
    y3 = y.astype(jnp.int32).reshape(_NB, _BR, 1)
    partial = pl.pallas_call(
        _dlr_body,
        grid=(_NB,),
        in_specs=[
            pl.BlockSpec((_BR, _C), lambda i: (i, 0)),
            pl.BlockSpec((1, _BR, 1), lambda i: (i, 0, 0)),
        ],
        out_specs=pl.BlockSpec((1, 1, 1), lambda i: (i, 0, 0)),
        out_shape=jax.ShapeDtypeStruct((_NB, 1, 1), jnp.float32),
        compiler_params=pltpu.CompilerParams(
            dimension_semantics=("parallel",),
        ),
    )(x, y3)
    return -(jnp.sum(partial) / _B)


# speedup vs baseline: 64.4422x; 2.4647x over previous
import jax
import jax.numpy as jnp
from jax.experimental import pallas as pl
from jax.experimental.pallas import tpu as pltpu

_B = 16384
_C = 1000
_BC = 512
_NB = _B // _BC

_NEG = float(-jnp.inf)


def _dlr_body(xt_ref, y_ref, out_ref):
    xb = xt_ref[...]
    yv = y_ref[0]

    gs = [xb[k * 128:(k + 1) * 128, :] for k in range(7)]
    rows = jax.lax.broadcasted_iota(jnp.int32, (128, _BC), 0)
    gs.append(jnp.where(rows >= 24, xb[872:1000, :], _NEG))

    hi = [jnp.maximum(gs[2 * i], gs[2 * i + 1]) for i in range(4)]
    lo = [jnp.minimum(gs[2 * i], gs[2 * i + 1]) for i in range(4)]

    def top3of4(a, b, c, d):
        p1 = jnp.maximum(a, c)
        p2 = jnp.minimum(a, c)
        q1 = jnp.maximum(b, d)
        return p1, jnp.maximum(p2, q1), jnp.minimum(p2, q1)

    x1, x2, x3 = top3of4(hi[0], lo[0], hi[1], lo[1])
    y1, y2, y3 = top3of4(hi[2], lo[2], hi[3], lo[3])
    z1 = jnp.maximum(x1, y1)
    m11 = jnp.minimum(x1, y1)
    m22 = jnp.maximum(x2, y2)
    z2 = jnp.maximum(m11, m22)
    z3 = jnp.maximum(jnp.maximum(jnp.minimum(m22, m11), jnp.minimum(x2, y2)),
                     jnp.maximum(x3, y3))

    cand = jnp.concatenate([z2, z3], axis=0)
    m1 = jnp.max(z1, axis=0, keepdims=True)
    c1z = z1 == m1
    c1c = cand == m1
    k1 = (jnp.sum(jnp.where(c1z, 1.0, 0.0), axis=0, keepdims=True)
          + jnp.sum(jnp.where(c1c, 1.0, 0.0), axis=0, keepdims=True))
    uz = jnp.where(c1z, _NEG, z1)
    uc = jnp.where(c1c, _NEG, cand)
    u = jnp.maximum(jnp.max(uz, axis=0, keepdims=True),
                    jnp.max(uc, axis=0, keepdims=True))
    c2z = uz == u
    c2c = uc == u
    k2 = (jnp.sum(jnp.where(c2z, 1.0, 0.0), axis=0, keepdims=True)
          + jnp.sum(jnp.where(c2c, 1.0, 0.0), axis=0, keepdims=True))
    v = jnp.maximum(
        jnp.max(jnp.where(c2z, _NEG, uz), axis=0, keepdims=True),
        jnp.max(jnp.where(c2c, _NEG, uc), axis=0, keepdims=True))
    m2 = jnp.where(k1 > 1.0, m1, u)
    m3 = jnp.where(k1 > 2.0, m1, jnp.where(k1 + k2 > 2.0, u, v))

    tacc = jnp.where(rows == yv, gs[0], _NEG)
    for k in range(1, 7):
        tacc = jnp.maximum(tacc, jnp.where(rows == yv - (128 * k), gs[k], _NEG))
    tacc = jnp.maximum(tacc, jnp.where(rows == yv - 872, gs[7], _NEG))
    tl = jnp.max(tacc, axis=0, keepdims=True)

    ind = tl == m1
    dividend = tl - jnp.where(ind, m2, m1)
    divisor = m1 - m3 + 1e-12
    out_ref[0] = jnp.sum(dividend / divisor, axis=1, keepdims=True)


def kernel(x, y):
    xt = x.T
    y3 = y.astype(jnp.int32).reshape(_NB, 1, _BC)
    partial = pl.pallas_call(
        _dlr_body,
        grid=(_NB,),
        in_specs=[
            pl.BlockSpec((_C, _BC), lambda i: (0, i)),
            pl.BlockSpec((1, 1, _BC), lambda i: (i, 0, 0)),
        ],
        out_specs=pl.BlockSpec((1, 1, 1), lambda i: (i, 0, 0)),
        out_shape=jax.ShapeDtypeStruct((_NB, 1, 1), jnp.float32),
        compiler_params=pltpu.CompilerParams(
            dimension_semantics=("parallel",),
        ),
    )(xt, y3)
    return -(jnp.sum(partial) / _B)


# speedup vs baseline: 68.9851x; 1.0705x over previous
import jax
import jax.numpy as jnp
from jax.experimental import pallas as pl
from jax.experimental.pallas import tpu as pltpu

_B = 16384
_C = 1000
_BC = 512
_NB = _B // _BC

_NEG = float(-jnp.inf)


def _dlr_body(xt_ref, y_ref, out_ref):
    yv = y_ref[0]

    gs = [xt_ref[k * 128:(k + 1) * 128, :] for k in range(7)]
    rows = jax.lax.broadcasted_iota(jnp.int32, (128, _BC), 0)
    gs.append(jnp.where(rows >= 24, xt_ref[872:1000, :], _NEG))

    hi = [jnp.maximum(gs[2 * i], gs[2 * i + 1]) for i in range(4)]
    lo = [jnp.minimum(gs[2 * i], gs[2 * i + 1]) for i in range(4)]

    def top3of4(a, b, c, d):
        p1 = jnp.maximum(a, c)
        p2 = jnp.minimum(a, c)
        q1 = jnp.maximum(b, d)
        return p1, jnp.maximum(p2, q1), jnp.minimum(p2, q1)

    x1, x2, x3 = top3of4(hi[0], lo[0], hi[1], lo[1])
    y1, y2, y3 = top3of4(hi[2], lo[2], hi[3], lo[3])
    z1 = jnp.maximum(x1, y1)
    m11 = jnp.minimum(x1, y1)
    m22 = jnp.maximum(x2, y2)
    z2 = jnp.maximum(m11, m22)
    z3 = jnp.maximum(jnp.maximum(jnp.minimum(m22, m11), jnp.minimum(x2, y2)),
                     jnp.maximum(x3, y3))

    m1 = jnp.max(z1, axis=0, keepdims=True)
    c1 = z1 == m1
    cnt1 = jnp.sum(jnp.where(c1, 1.0, 0.0), axis=0, keepdims=True)
    b = jnp.where(c1, z2, z1)
    m2r = jnp.max(b, axis=0, keepdims=True)
    cb = b == m2r
    cnt2 = jnp.sum(jnp.where(cb, 1.0, 0.0), axis=0, keepdims=True)
    c = jnp.where(cb, jnp.where(c1, z3, z2), b)
    m3r = jnp.max(c, axis=0, keepdims=True)
    m2 = jnp.where(cnt1 > 1.0, m1, m2r)
    m3 = jnp.where(cnt1 > 2.0, m1,
                   jnp.where(cnt1 > 1.0, m2r,
                             jnp.where(cnt2 > 1.0, m2r, m3r)))

    tacc = jnp.where(rows == yv, gs[0], _NEG)
    for k in range(1, 7):
        tacc = jnp.maximum(tacc, jnp.where(rows == yv - (128 * k), gs[k], _NEG))
    tacc = jnp.maximum(tacc, jnp.where(rows == yv - 872, gs[7], _NEG))
    tl = jnp.max(tacc, axis=0, keepdims=True)

    ind = tl == m1
    dividend = tl - jnp.where(ind, m2, m1)
    divisor = m1 - m3 + 1e-12
    out_ref[0] = jnp.sum(dividend / divisor, axis=1, keepdims=True)


def kernel(x, y):
    xt = x.T
    y3 = y.astype(jnp.int32).reshape(_NB, 1, _BC)
    partial = pl.pallas_call(
        _dlr_body,
        grid=(_NB,),
        in_specs=[
            pl.BlockSpec((_C, _BC), lambda i: (0, i)),
            pl.BlockSpec((1, 1, _BC), lambda i: (i, 0, 0)),
        ],
        out_specs=pl.BlockSpec((1, 1, 1), lambda i: (i, 0, 0)),
        out_shape=jax.ShapeDtypeStruct((_NB, 1, 1), jnp.float32),
        compiler_params=pltpu.CompilerParams(
            dimension_semantics=("parallel",),
        ),
    )(xt, y3)
    return -(jnp.sum(partial) / _B)


# speedup vs baseline: 69.5770x; 1.0086x over previous
import jax
import jax.numpy as jnp
from jax.experimental import pallas as pl
from jax.experimental.pallas import tpu as pltpu

_B = 16384
_C = 1000
_BC = 512
_NB = _B // _BC

_NEG = float(-jnp.inf)


def _dlr_body(xt_ref, y_ref, out_ref):
    yv = y_ref[0]

    gs = [xt_ref[k * 128:(k + 1) * 128, :] for k in range(7)]
    rows = jax.lax.broadcasted_iota(jnp.int32, (128, _BC), 0)
    gs.append(jnp.where(rows >= 24, xt_ref[872:1000, :], _NEG))

    hi = [jnp.maximum(gs[2 * i], gs[2 * i + 1]) for i in range(4)]
    lo = [jnp.minimum(gs[2 * i], gs[2 * i + 1]) for i in range(4)]

    def top3of4(a, b, c, d):
        p1 = jnp.maximum(a, c)
        p2 = jnp.minimum(a, c)
        q1 = jnp.maximum(b, d)
        return p1, jnp.maximum(p2, q1), jnp.minimum(p2, q1)

    x1, x2, x3 = top3of4(hi[0], lo[0], hi[1], lo[1])
    y1, y2, y3 = top3of4(hi[2], lo[2], hi[3], lo[3])
    z1 = jnp.maximum(x1, y1)
    m11 = jnp.minimum(x1, y1)
    m22 = jnp.maximum(x2, y2)
    z2 = jnp.maximum(m11, m22)
    z3 = jnp.maximum(jnp.maximum(jnp.minimum(m22, m11), jnp.minimum(x2, y2)),
                     jnp.maximum(x3, y3))

    m1 = jnp.max(z1, axis=0, keepdims=True)
    c1 = z1 == m1
    cnt1 = jnp.sum(jnp.where(c1, 1.0, 0.0), axis=0, keepdims=True)
    b = jnp.where(c1, z2, z1)
    m2r = jnp.max(b, axis=0, keepdims=True)
    cb = b == m2r
    cnt2 = jnp.sum(jnp.where(cb, 1.0, 0.0), axis=0, keepdims=True)
    c = jnp.where(cb, jnp.where(c1, z3, z2), b)
    m3r = jnp.max(c, axis=0, keepdims=True)
    m2 = jnp.where(cnt1 > 1.0, m1, m2r)
    m3 = jnp.where(cnt1 > 2.0, m1,
                   jnp.where(cnt1 > 1.0, m2r,
                             jnp.where(cnt2 > 1.0, m2r, m3r)))

    g7t = jnp.concatenate(
        [xt_ref[896:1000, :], jnp.full((24, _BC), _NEG, jnp.float32)], axis=0)
    sid = jax.lax.shift_right_logical(yv, 7)
    b0 = (sid & 1) == 1
    b1 = (jax.lax.shift_right_logical(sid, 1) & 1) == 1
    b2 = jax.lax.shift_right_logical(sid, 2) == 1
    t01 = jnp.where(b0, gs[1], gs[0])
    t23 = jnp.where(b0, gs[3], gs[2])
    t45 = jnp.where(b0, gs[5], gs[4])
    t67 = jnp.where(b0, g7t, gs[6])
    t03 = jnp.where(b1, t23, t01)
    t47 = jnp.where(b1, t67, t45)
    g_y = jnp.where(b2, t47, t03)
    oh = rows == (yv & 127)
    tl = jnp.max(jnp.where(oh, g_y, _NEG), axis=0, keepdims=True)

    ind = tl == m1
    dividend = tl - jnp.where(ind, m2, m1)
    divisor = m1 - m3 + 1e-12
    out_ref[0] = jnp.sum(dividend / divisor, axis=1, keepdims=True)


def kernel(x, y):
    xt = x.T
    y3 = y.astype(jnp.int32).reshape(_NB, 1, _BC)
    partial = pl.pallas_call(
        _dlr_body,
        grid=(_NB,),
        in_specs=[
            pl.BlockSpec((_C, _BC), lambda i: (0, i)),
            pl.BlockSpec((1, 1, _BC), lambda i: (i, 0, 0)),
        ],
        out_specs=pl.BlockSpec((1, 1, 1), lambda i: (i, 0, 0)),
        out_shape=jax.ShapeDtypeStruct((_NB, 1, 1), jnp.float32),
        compiler_params=pltpu.CompilerParams(
            dimension_semantics=("parallel",),
        ),
    )(xt, y3)
    return -(jnp.sum(partial) / _B)
